# Initial kernel scaffold; baseline (speedup 1.0000x reference)
import jax
import jax.numpy as jnp
from jax.experimental import pallas as pl


def kernel(w1, b1, w12, b12, w2, b2, w3, b3, w4, b4, lw, lb, fw, fb, ow, ob, x):
    raise NotImplementedError("write your pallas kernel here")


# speedup vs baseline: 1.5694x; 1.5694x over previous
import jax
import jax.numpy as jnp
from jax.experimental import pallas as pl
from jax.experimental.pallas import tpu as pltpu

_F32 = jnp.float32
_BF16 = jnp.bfloat16


def _conv3x3(xp, wk, b, H, W, Cout):
    Wp = xp.shape[1]
    xcat = jnp.concatenate([xp[0:H], xp[1:H + 1], xp[2:H + 2]], axis=-1)
    z = jnp.dot(xcat.reshape(H * Wp, xcat.shape[-1]), wk,
                preferred_element_type=_F32).reshape(H, Wp, 3 * Cout)
    y = (z[:, 0:W, 0:Cout] + z[:, 1:W + 1, Cout:2 * Cout]
         + z[:, 2:W + 2, 2 * Cout:3 * Cout] + b)
    return jnp.maximum(y, 0.0)


def _pool2x2(st, H, W, C):
    ev = st[:, pl.ds(0, W // 2, 2), :]
    od = st[:, pl.ds(1, W // 2, 2), :]
    m = jnp.maximum(ev, od).reshape(H // 2, 2, W // 2, C)
    return jnp.maximum(m[:, 0], m[:, 1])


def _conv_kernel(x_ref, w1_ref, b1_ref, w12_ref, b12_ref, w2_ref, b2_ref,
                 w3_ref, b3_ref, w4_ref, b4_ref, o_ref,
                 a1, a2, a3, a4, s12, s2, s4):
    a1[...] = jnp.zeros_like(a1)
    a2[...] = jnp.zeros_like(a2)
    a3[...] = jnp.zeros_like(a3)
    a4[...] = jnp.zeros_like(a4)

    xp = x_ref[0]
    y1 = _conv3x3(xp, w1_ref[...], b1_ref[...], 64, 64, 32)
    a1[1:65, 1:65, :] = y1.astype(_BF16)

    y12 = _conv3x3(a1[...], w12_ref[...], b12_ref[...], 64, 64, 32)
    s12[...] = y12
    a2[1:33, 1:33, :] = _pool2x2(s12, 64, 64, 32).astype(_BF16)

    y2 = _conv3x3(a2[...], w2_ref[...], b2_ref[...], 32, 32, 32)
    s2[...] = y2
    a3[1:17, 1:17, :] = _pool2x2(s2, 32, 32, 32).astype(_BF16)

    y3 = _conv3x3(a3[...], w3_ref[...], b3_ref[...], 16, 16, 64)
    a4[1:17, 1:17, :] = y3.astype(_BF16)

    y4 = _conv3x3(a4[...], w4_ref[...], b4_ref[...], 16, 16, 64)
    s4[...] = y4
    o_ref[0] = _pool2x2(s4, 16, 16, 64).astype(_BF16)


def _mlp_kernel(f_ref, lw_ref, lb_ref, fw_ref, fb_ref, ow_ref, ob_ref, o_ref):
    h = jnp.dot(f_ref[...], lw_ref[...], preferred_element_type=_F32)
    h = jnp.maximum(h + lb_ref[...], 0.0).astype(_BF16)
    h = jnp.dot(h, fw_ref[...], preferred_element_type=_F32)
    h = jnp.maximum(h + fb_ref[...], 0.0).astype(_BF16)
    o_ref[...] = jnp.dot(h, ow_ref[...], preferred_element_type=_F32) + ob_ref[...]


def _wk(w, cin, cout):
    return (w.reshape(3, 3, cin, cout).transpose(0, 2, 1, 3)
            .reshape(3 * cin, 3 * cout).astype(_BF16))


def kernel(w1, b1, w12, b12, w2, b2, w3, b3, w4, b4,
           lw, lb, fw, fb, ow, ob, x):
    B = x.shape[0]
    xh = jnp.transpose(x, (0, 2, 3, 1))
    xp = jnp.pad(xh, ((0, 0), (1, 1), (1, 15), (0, 5))).astype(_BF16)

    w1k = _wk(w1, 8, 32)
    w12k = _wk(w12, 32, 32)
    w2k = _wk(w2, 32, 32)
    w3k = _wk(w3, 32, 64)
    w4k = _wk(w4, 64, 64)

    feats = pl.pallas_call(
        _conv_kernel,
        out_shape=jax.ShapeDtypeStruct((B, 8, 8, 64), _BF16),
        grid=(B,),
        in_specs=[
            pl.BlockSpec((1, 66, 80, 8), lambda b: (b, 0, 0, 0)),
            pl.BlockSpec((24, 96), lambda b: (0, 0)),
            pl.BlockSpec((1, 32), lambda b: (0, 0)),
            pl.BlockSpec((96, 96), lambda b: (0, 0)),
            pl.BlockSpec((1, 32), lambda b: (0, 0)),
            pl.BlockSpec((96, 96), lambda b: (0, 0)),
            pl.BlockSpec((1, 32), lambda b: (0, 0)),
            pl.BlockSpec((96, 192), lambda b: (0, 0)),
            pl.BlockSpec((1, 64), lambda b: (0, 0)),
            pl.BlockSpec((192, 192), lambda b: (0, 0)),
            pl.BlockSpec((1, 64), lambda b: (0, 0)),
        ],
        out_specs=pl.BlockSpec((1, 8, 8, 64), lambda b: (b, 0, 0, 0)),
        scratch_shapes=[
            pltpu.VMEM((66, 80, 32), _BF16),
            pltpu.VMEM((34, 48, 32), _BF16),
            pltpu.VMEM((18, 32, 32), _BF16),
            pltpu.VMEM((18, 32, 64), _BF16),
            pltpu.VMEM((64, 64, 32), _F32),
            pltpu.VMEM((32, 32, 32), _F32),
            pltpu.VMEM((16, 16, 64), _F32),
        ],
        compiler_params=pltpu.CompilerParams(
            dimension_semantics=("parallel",),
            vmem_limit_bytes=48 * 1024 * 1024,
        ),
    )(xp, w1k, b1, w12k, b12, w2k, b2, w3k, b3, w4k, b4)

    flat = feats.reshape(B, 4096)
    TM = 256 if B % 256 == 0 else B
    out = pl.pallas_call(
        _mlp_kernel,
        out_shape=jax.ShapeDtypeStruct((B, 10), _F32),
        grid=(B // TM,),
        in_specs=[
            pl.BlockSpec((TM, 4096), lambda i: (i, 0)),
            pl.BlockSpec((4096, 256), lambda i: (0, 0)),
            pl.BlockSpec((1, 256), lambda i: (0, 0)),
            pl.BlockSpec((256, 256), lambda i: (0, 0)),
            pl.BlockSpec((1, 256), lambda i: (0, 0)),
            pl.BlockSpec((256, 10), lambda i: (0, 0)),
            pl.BlockSpec((1, 10), lambda i: (0, 0)),
        ],
        out_specs=pl.BlockSpec((TM, 10), lambda i: (i, 0)),
        compiler_params=pltpu.CompilerParams(
            dimension_semantics=("parallel",),
        ),
    )(flat, lw.astype(_BF16), lb, fw.astype(_BF16), fb, ow.astype(_BF16), ob)
    return out


# speedup vs baseline: 3.5104x; 2.2367x over previous
import jax
import jax.numpy as jnp
from jax.experimental import pallas as pl
from jax.experimental.pallas import tpu as pltpu

_F32 = jnp.float32
_BF16 = jnp.bfloat16


def _conv_kernel(x_ref, wt1_ref, wt12_ref, wt2_ref, wt3_ref, wt4_ref,
                 m80_ref, m32_ref, o_ref,
                 rhs, x12, x2, x3, x4, s1, sb1, s2p, sb2, s4p, sb4, p2, p3):
    def load_taps(xv, cin, stride, n):
        for t in range(9):
            dh, dw = divmod(t, 3)
            off = dh * stride + dw
            rhs[t * cin:(t + 1) * cin, 0:n] = xv[:, off:off + n]
        rhs[9 * cin:9 * cin + 1, 0:n] = jnp.ones((1, n), _BF16)

    load_taps(x_ref[0], 3, 80, 5120)
    z = jnp.dot(wt1_ref[...], rhs[0:28, 0:5120], preferred_element_type=_F32)
    y1 = jnp.maximum(z, 0.0) * m80_ref[...]
    x12[...] = jnp.zeros_like(x12)
    x12[:, 81:5201] = y1.astype(_BF16)

    load_taps(x12[...], 32, 80, 5120)
    z = jnp.dot(wt12_ref[...], rhs[0:289, 0:5120], preferred_element_type=_F32)
    s1[...] = jnp.transpose(jnp.maximum(z, 0.0))
    a = jnp.maximum(s1[pl.ds(0, 2560, 2), :], s1[pl.ds(1, 2560, 2), :])
    sb1[...] = a.reshape(64, 40, 32)
    pooled = jnp.maximum(sb1[pl.ds(0, 32, 2)], sb1[pl.ds(1, 32, 2)])[:, 0:32, :]
    p2[...] = jnp.zeros_like(p2)
    p2[1:33, 1:33, :] = pooled.astype(_BF16)
    x2[...] = jnp.zeros_like(x2)
    x2[:, 0:1632] = jnp.transpose(p2[...].reshape(1632, 32))

    load_taps(x2[...], 32, 48, 1536)
    z = jnp.dot(wt2_ref[...], rhs[0:289, 0:1536], preferred_element_type=_F32)
    s2p[...] = jnp.transpose(jnp.maximum(z, 0.0))
    a = jnp.maximum(s2p[pl.ds(0, 768, 2), :], s2p[pl.ds(1, 768, 2), :])
    sb2[...] = a.reshape(32, 24, 32)
    pooled = jnp.maximum(sb2[pl.ds(0, 16, 2)], sb2[pl.ds(1, 16, 2)])[:, 0:16, :]
    p3[...] = jnp.zeros_like(p3)
    p3[1:17, 1:17, :] = pooled.astype(_BF16)
    x3[...] = jnp.zeros_like(x3)
    x3[:, 0:576] = jnp.transpose(p3[...].reshape(576, 32))

    load_taps(x3[...], 32, 32, 512)
    z = jnp.dot(wt3_ref[...], rhs[0:289, 0:512], preferred_element_type=_F32)
    y3 = jnp.maximum(z, 0.0) * m32_ref[...]
    x4[...] = jnp.zeros_like(x4)
    x4[:, 33:545] = y3.astype(_BF16)

    load_taps(x4[...], 64, 32, 512)
    z = jnp.dot(wt4_ref[...], rhs[0:577, 0:512], preferred_element_type=_F32)
    s4p[...] = jnp.transpose(jnp.maximum(z, 0.0))
    a = jnp.maximum(s4p[pl.ds(0, 256, 2), :], s4p[pl.ds(1, 256, 2), :])
    sb4[...] = a.reshape(16, 16, 64)
    pooled = jnp.maximum(sb4[pl.ds(0, 8, 2)], sb4[pl.ds(1, 8, 2)])[:, 0:8, :]
    o_ref[0] = pooled.astype(_BF16)


def _mlp_kernel(f_ref, lw_ref, lb_ref, fw_ref, fb_ref, ow_ref, ob_ref, o_ref):
    h = jnp.dot(f_ref[...].astype(_F32), lw_ref[...],
                preferred_element_type=_F32)
    h = jnp.maximum(h + lb_ref[...], 0.0)
    h = jnp.dot(h, fw_ref[...], preferred_element_type=_F32)
    h = jnp.maximum(h + fb_ref[...], 0.0)
    o_ref[...] = jnp.dot(h, ow_ref[...], preferred_element_type=_F32) + ob_ref[...]


def _wt(w, cin, cout, b):
    m = (w.reshape(3, 3, cin, cout).transpose(3, 0, 1, 2).reshape(cout, 9 * cin))
    return jnp.concatenate([m, b.reshape(cout, 1)], axis=1).astype(_BF16)


def kernel(w1, b1, w12, b12, w2, b2, w3, b3, w4, b4,
           lw, lb, fw, fb, ow, ob, x):
    B = x.shape[0]
    xp = jnp.pad(x, ((0, 0), (0, 0), (1, 1), (1, 15)))
    xf = jnp.pad(xp.reshape(B, 3, 5280),
                 ((0, 0), (0, 0), (0, 16))).astype(_BF16)

    wt1 = _wt(w1.reshape(3, 3, 8, 32)[:, :, 0:3, :].reshape(3, 9, 32), 3, 32, b1)
    wt12 = _wt(w12, 32, 32, b12)
    wt2 = _wt(w2, 32, 32, b2)
    wt3 = _wt(w3, 32, 64, b3)
    wt4 = _wt(w4, 64, 64, b4)

    m80 = ((jnp.arange(5120) % 80) < 64).astype(_F32).reshape(1, 5120)
    m32 = ((jnp.arange(512) % 32) < 16).astype(_F32).reshape(1, 512)

    feats = pl.pallas_call(
        _conv_kernel,
        out_shape=jax.ShapeDtypeStruct((B, 8, 8, 64), _BF16),
        grid=(B,),
        in_specs=[
            pl.BlockSpec((1, 3, 5296), lambda b: (b, 0, 0)),
            pl.BlockSpec((32, 28), lambda b: (0, 0)),
            pl.BlockSpec((32, 289), lambda b: (0, 0)),
            pl.BlockSpec((32, 289), lambda b: (0, 0)),
            pl.BlockSpec((64, 289), lambda b: (0, 0)),
            pl.BlockSpec((64, 577), lambda b: (0, 0)),
            pl.BlockSpec((1, 5120), lambda b: (0, 0)),
            pl.BlockSpec((1, 512), lambda b: (0, 0)),
        ],
        out_specs=pl.BlockSpec((1, 8, 8, 64), lambda b: (b, 0, 0, 0)),
        scratch_shapes=[
            pltpu.VMEM((592, 5120), _BF16),
            pltpu.VMEM((32, 5296), _BF16),
            pltpu.VMEM((32, 1648), _BF16),
            pltpu.VMEM((32, 592), _BF16),
            pltpu.VMEM((64, 592), _BF16),
            pltpu.VMEM((5120, 32), _F32),
            pltpu.VMEM((64, 40, 32), _F32),
            pltpu.VMEM((1536, 32), _F32),
            pltpu.VMEM((32, 24, 32), _F32),
            pltpu.VMEM((512, 64), _F32),
            pltpu.VMEM((16, 16, 64), _F32),
            pltpu.VMEM((34, 48, 32), _BF16),
            pltpu.VMEM((18, 32, 32), _BF16),
        ],
        compiler_params=pltpu.CompilerParams(
            dimension_semantics=("parallel",),
            vmem_limit_bytes=48 * 1024 * 1024,
        ),
    )(xf, wt1, wt12, wt2, wt3, wt4, m80, m32)

    flat = feats.reshape(B, 4096)
    TM = 256 if B % 256 == 0 else B
    out = pl.pallas_call(
        _mlp_kernel,
        out_shape=jax.ShapeDtypeStruct((B, 10), _F32),
        grid=(B // TM,),
        in_specs=[
            pl.BlockSpec((TM, 4096), lambda i: (i, 0)),
            pl.BlockSpec((4096, 256), lambda i: (0, 0)),
            pl.BlockSpec((1, 256), lambda i: (0, 0)),
            pl.BlockSpec((256, 256), lambda i: (0, 0)),
            pl.BlockSpec((1, 256), lambda i: (0, 0)),
            pl.BlockSpec((256, 10), lambda i: (0, 0)),
            pl.BlockSpec((1, 10), lambda i: (0, 0)),
        ],
        out_specs=pl.BlockSpec((TM, 10), lambda i: (i, 0)),
        compiler_params=pltpu.CompilerParams(
            dimension_semantics=("parallel",),
        ),
    )(flat, lw, lb, fw, fb, ow, ob)
    return out


# speedup vs baseline: 3.5792x; 1.0196x over previous
import jax
import jax.numpy as jnp
from jax.experimental import pallas as pl
from jax.experimental.pallas import tpu as pltpu

_F32 = jnp.float32
_BF16 = jnp.bfloat16


def _conv_kernel(x_ref, wt1_ref, wt12_ref, wt2_ref, wt3_ref, wt4_ref,
                 m80_ref, m32_ref, o_ref,
                 rhs, x12, x2, x3, x4, s1, s2p, s4p, p2, p3):
    def load_taps(xv, cin, stride, n):
        for t in range(9):
            dh, dw = divmod(t, 3)
            off = dh * stride + dw
            rhs[t * cin:(t + 1) * cin, 0:n] = xv[:, off:off + n]
        rhs[9 * cin:9 * cin + 1, 0:n] = jnp.ones((1, n), _BF16)

    def pool_pairs(s, n2, hh, ww):
        a = jnp.maximum(s[pl.ds(0, n2, 2), :], s[pl.ds(1, n2, 2), :])
        m = a.astype(_F32).reshape(hh // 2, 2, ww, a.shape[-1])
        return jnp.maximum(m[:, 0], m[:, 1])

    load_taps(x_ref[0], 3, 80, 5120)
    z = jnp.dot(wt1_ref[...], rhs[0:28, 0:5120], preferred_element_type=_F32)
    y1 = jnp.maximum(z, 0.0) * m80_ref[...]
    x12[:, 81:5201] = y1.astype(_BF16)
    x12[:, 0:81] = jnp.zeros((32, 81), _BF16)
    x12[:, 5201:5296] = jnp.zeros((32, 95), _BF16)

    load_taps(x12[...], 32, 80, 5120)
    z = jnp.dot(wt12_ref[...], rhs[0:289, 0:5120], preferred_element_type=_F32)
    s1[...] = jnp.transpose(jnp.maximum(z, 0.0))
    pooled = pool_pairs(s1, 2560, 64, 40)[:, 0:32, :]
    p2[...] = jnp.zeros_like(p2)
    p2[1:33, 1:33, :] = pooled.astype(_BF16)
    x2[:, 0:1632] = jnp.transpose(p2[...].reshape(1632, 32))
    x2[:, 1632:1648] = jnp.zeros((32, 16), _BF16)

    load_taps(x2[...], 32, 48, 1536)
    z = jnp.dot(wt2_ref[...], rhs[0:289, 0:1536], preferred_element_type=_F32)
    s2p[...] = jnp.transpose(jnp.maximum(z, 0.0))
    pooled = pool_pairs(s2p, 768, 32, 24)[:, 0:16, :]
    p3[...] = jnp.zeros_like(p3)
    p3[1:17, 1:17, :] = pooled.astype(_BF16)
    x3[:, 0:576] = jnp.transpose(p3[...].reshape(576, 32))
    x3[:, 576:592] = jnp.zeros((32, 16), _BF16)

    load_taps(x3[...], 32, 32, 512)
    z = jnp.dot(wt3_ref[...], rhs[0:289, 0:512], preferred_element_type=_F32)
    y3 = jnp.maximum(z, 0.0) * m32_ref[...]
    x4[:, 33:545] = y3.astype(_BF16)
    x4[:, 0:33] = jnp.zeros((64, 33), _BF16)
    x4[:, 545:592] = jnp.zeros((64, 47), _BF16)

    load_taps(x4[...], 64, 32, 512)
    z = jnp.dot(wt4_ref[...], rhs[0:577, 0:512], preferred_element_type=_F32)
    s4p[...] = jnp.transpose(jnp.maximum(z, 0.0))
    pooled = pool_pairs(s4p, 256, 16, 16)[:, 0:8, :]
    o_ref[0] = pooled.astype(_BF16)


def _mlp_kernel(f_ref, lw_ref, lb_ref, fw_ref, fb_ref, ow_ref, ob_ref, o_ref):
    h = jnp.dot(f_ref[...].astype(_F32), lw_ref[...],
                preferred_element_type=_F32)
    h = jnp.maximum(h + lb_ref[...], 0.0)
    h = jnp.dot(h, fw_ref[...], preferred_element_type=_F32)
    h = jnp.maximum(h + fb_ref[...], 0.0)
    o_ref[...] = jnp.dot(h, ow_ref[...], preferred_element_type=_F32) + ob_ref[...]


def _wt(w, cin, cout, b):
    m = (w.reshape(3, 3, cin, cout).transpose(3, 0, 1, 2).reshape(cout, 9 * cin))
    return jnp.concatenate([m, b.reshape(cout, 1)], axis=1).astype(_BF16)


def kernel(w1, b1, w12, b12, w2, b2, w3, b3, w4, b4,
           lw, lb, fw, fb, ow, ob, x):
    B = x.shape[0]
    xp = jnp.pad(x, ((0, 0), (0, 0), (1, 1), (1, 15)))
    xf = jnp.pad(xp.reshape(B, 3, 5280),
                 ((0, 0), (0, 0), (0, 16))).astype(_BF16)

    wt1 = _wt(w1.reshape(3, 3, 8, 32)[:, :, 0:3, :].reshape(3, 9, 32), 3, 32, b1)
    wt12 = _wt(w12, 32, 32, b12)
    wt2 = _wt(w2, 32, 32, b2)
    wt3 = _wt(w3, 32, 64, b3)
    wt4 = _wt(w4, 64, 64, b4)

    m80 = ((jnp.arange(5120) % 80) < 64).astype(_F32).reshape(1, 5120)
    m32 = ((jnp.arange(512) % 32) < 16).astype(_F32).reshape(1, 512)

    HB = B // 2
    feats = pl.pallas_call(
        _conv_kernel,
        out_shape=jax.ShapeDtypeStruct((B, 8, 8, 64), _BF16),
        grid=(2, HB),
        in_specs=[
            pl.BlockSpec((1, 3, 5296), lambda c, b: (c * HB + b, 0, 0)),
            pl.BlockSpec((32, 28), lambda c, b: (0, 0)),
            pl.BlockSpec((32, 289), lambda c, b: (0, 0)),
            pl.BlockSpec((32, 289), lambda c, b: (0, 0)),
            pl.BlockSpec((64, 289), lambda c, b: (0, 0)),
            pl.BlockSpec((64, 577), lambda c, b: (0, 0)),
            pl.BlockSpec((1, 5120), lambda c, b: (0, 0)),
            pl.BlockSpec((1, 512), lambda c, b: (0, 0)),
        ],
        out_specs=pl.BlockSpec((1, 8, 8, 64),
                               lambda c, b: (c * HB + b, 0, 0, 0)),
        scratch_shapes=[
            pltpu.VMEM((592, 5120), _BF16),
            pltpu.VMEM((32, 5296), _BF16),
            pltpu.VMEM((32, 1648), _BF16),
            pltpu.VMEM((32, 592), _BF16),
            pltpu.VMEM((64, 592), _BF16),
            pltpu.VMEM((5120, 32), _F32),
            pltpu.VMEM((1536, 32), _F32),
            pltpu.VMEM((512, 64), _F32),
            pltpu.VMEM((34, 48, 32), _BF16),
            pltpu.VMEM((18, 32, 32), _BF16),
        ],
        compiler_params=pltpu.CompilerParams(
            dimension_semantics=("parallel", "arbitrary"),
            vmem_limit_bytes=48 * 1024 * 1024,
        ),
    )(xf, wt1, wt12, wt2, wt3, wt4, m80, m32)

    flat = feats.reshape(B, 4096)
    TM = 256 if B % 256 == 0 else B
    out = pl.pallas_call(
        _mlp_kernel,
        out_shape=jax.ShapeDtypeStruct((B, 10), _F32),
        grid=(B // TM,),
        in_specs=[
            pl.BlockSpec((TM, 4096), lambda i: (i, 0)),
            pl.BlockSpec((4096, 256), lambda i: (0, 0)),
            pl.BlockSpec((1, 256), lambda i: (0, 0)),
            pl.BlockSpec((256, 256), lambda i: (0, 0)),
            pl.BlockSpec((1, 256), lambda i: (0, 0)),
            pl.BlockSpec((256, 10), lambda i: (0, 0)),
            pl.BlockSpec((1, 10), lambda i: (0, 0)),
        ],
        out_specs=pl.BlockSpec((TM, 10), lambda i: (i, 0)),
        compiler_params=pltpu.CompilerParams(
            dimension_semantics=("parallel",),
        ),
    )(flat, lw, lb, fw, fb, ow, ob)
    return out


# speedup vs baseline: 4.7596x; 1.3298x over previous
import jax
import jax.numpy as jnp
from jax.experimental import pallas as pl
from jax.experimental.pallas import tpu as pltpu

_F32 = jnp.float32
_BF16 = jnp.bfloat16


def _make_conv_kernel(G):
    def conv_kernel(x_ref, wt1_ref, wt12_ref, wt2_ref, wt3_ref, wt4_ref,
                    m80_ref, m32_ref, o_ref,
                    rhs, rhs4, x12, x2, x3, x4, s1, s2p, s4p, p2, p3):
        def load_taps(r, getplane, cin, stride, seg, n):
            for g in range(G):
                xv = getplane(g)
                for t in range(9):
                    dh, dw = divmod(t, 3)
                    off = g * seg + dh * stride + dw
                    r[t * cin:(t + 1) * cin, g * n:(g + 1) * n] = \
                        xv[:, off:off + n]
            r[9 * cin:9 * cin + 1, 0:G * n] = jnp.ones((1, G * n), _BF16)

        def pool_pairs(s, n2, blk, ww):
            a = jnp.maximum(s[pl.ds(0, n2, 2), :], s[pl.ds(1, n2, 2), :])
            m = a.astype(_F32).reshape(blk, 2, ww, a.shape[-1])
            return jnp.maximum(m[:, 0], m[:, 1])

        xall = x_ref[...]
        load_taps(rhs, lambda g: xall[g], 3, 80, 0, 5120)
        z = jnp.dot(wt1_ref[...], rhs[0:28, :], preferred_element_type=_F32)
        y1 = jnp.maximum(z, 0.0) * m80_ref[...]
        yb = y1.astype(_BF16)
        for g in range(G):
            q = g * 5296
            x12[:, q + 81:q + 5201] = yb[:, g * 5120:(g + 1) * 5120]
            x12[:, q:q + 81] = jnp.zeros((32, 81), _BF16)
            x12[:, q + 5201:q + 5296] = jnp.zeros((32, 95), _BF16)

        xv12 = x12[...]
        load_taps(rhs, lambda g: xv12, 32, 80, 5296, 5120)
        z = jnp.dot(wt12_ref[...], rhs[0:289, :], preferred_element_type=_F32)
        s1[...] = jnp.transpose(jnp.maximum(z, 0.0))
        pooled = pool_pairs(s1, G * 2560, G * 32, 40)[:, 0:32, :]
        p2[...] = jnp.zeros_like(p2)
        pb = pooled.astype(_BF16)
        for g in range(G):
            p2[g * 34 + 1:g * 34 + 33, 1:33, :] = pb[g * 32:(g + 1) * 32]
        t2 = jnp.transpose(p2[...].reshape(G * 1632, 32))
        for g in range(G):
            x2[:, g * 1648:g * 1648 + 1632] = t2[:, g * 1632:(g + 1) * 1632]
            x2[:, g * 1648 + 1632:(g + 1) * 1648] = jnp.zeros((32, 16), _BF16)

        xv2 = x2[...]
        load_taps(rhs, lambda g: xv2, 32, 48, 1648, 1536)
        z = jnp.dot(wt2_ref[...], rhs[0:289, 0:G * 1536],
                    preferred_element_type=_F32)
        s2p[...] = jnp.transpose(jnp.maximum(z, 0.0))
        pooled = pool_pairs(s2p, G * 768, G * 16, 24)[:, 0:16, :]
        p3[...] = jnp.zeros_like(p3)
        pb = pooled.astype(_BF16)
        for g in range(G):
            p3[g * 18 + 1:g * 18 + 17, 1:17, :] = pb[g * 16:(g + 1) * 16]
        t3 = jnp.transpose(p3[...].reshape(G * 576, 32))
        for g in range(G):
            x3[:, g * 592:g * 592 + 576] = t3[:, g * 576:(g + 1) * 576]
            x3[:, g * 592 + 576:(g + 1) * 592] = jnp.zeros((32, 16), _BF16)

        xv3 = x3[...]
        load_taps(rhs4, lambda g: xv3, 32, 32, 592, 512)
        z = jnp.dot(wt3_ref[...], rhs4[0:289, :], preferred_element_type=_F32)
        y3 = jnp.maximum(z, 0.0) * m32_ref[...]
        yb = y3.astype(_BF16)
        for g in range(G):
            q = g * 592
            x4[:, q + 33:q + 545] = yb[:, g * 512:(g + 1) * 512]
            x4[:, q:q + 33] = jnp.zeros((64, 33), _BF16)
            x4[:, q + 545:q + 592] = jnp.zeros((64, 47), _BF16)

        xv4 = x4[...]
        load_taps(rhs4, lambda g: xv4, 64, 32, 592, 512)
        z = jnp.dot(wt4_ref[...], rhs4[0:577, :], preferred_element_type=_F32)
        s4p[...] = jnp.transpose(jnp.maximum(z, 0.0))
        pooled = pool_pairs(s4p, G * 256, G * 8, 16)[:, 0:8, :]
        o_ref[...] = pooled.astype(_BF16).reshape(G, 8, 8, 64)

    return conv_kernel


def _mlp_kernel(f_ref, lw_ref, lb_ref, fw_ref, fb_ref, ow_ref, ob_ref, o_ref):
    h = jnp.dot(f_ref[...].astype(_F32), lw_ref[...],
                preferred_element_type=_F32)
    h = jnp.maximum(h + lb_ref[...], 0.0)
    h = jnp.dot(h, fw_ref[...], preferred_element_type=_F32)
    h = jnp.maximum(h + fb_ref[...], 0.0)
    o_ref[...] = jnp.dot(h, ow_ref[...], preferred_element_type=_F32) + ob_ref[...]


def _wt(w, cin, cout, b):
    m = (w.reshape(3, 3, cin, cout).transpose(3, 0, 1, 2).reshape(cout, 9 * cin))
    return jnp.concatenate([m, b.reshape(cout, 1)], axis=1).astype(_BF16)


def kernel(w1, b1, w12, b12, w2, b2, w3, b3, w4, b4,
           lw, lb, fw, fb, ow, ob, x):
    B = x.shape[0]
    G = 4 if B % 4 == 0 else 1
    xp = jnp.pad(x, ((0, 0), (0, 0), (1, 1), (1, 15)))
    xf = jnp.pad(xp.reshape(B, 3, 5280),
                 ((0, 0), (0, 0), (0, 16))).astype(_BF16)

    wt1 = _wt(w1.reshape(3, 3, 8, 32)[:, :, 0:3, :].reshape(3, 9, 32), 3, 32, b1)
    wt12 = _wt(w12, 32, 32, b12)
    wt2 = _wt(w2, 32, 32, b2)
    wt3 = _wt(w3, 32, 64, b3)
    wt4 = _wt(w4, 64, 64, b4)

    m80 = jnp.tile(((jnp.arange(5120) % 80) < 64).astype(_F32), (G,)
                   ).reshape(1, G * 5120)
    m32 = jnp.tile(((jnp.arange(512) % 32) < 16).astype(_F32), (G,)
                   ).reshape(1, G * 512)

    feats = pl.pallas_call(
        _make_conv_kernel(G),
        out_shape=jax.ShapeDtypeStruct((B, 8, 8, 64), _BF16),
        grid=(B // G,),
        in_specs=[
            pl.BlockSpec((G, 3, 5296), lambda b: (b, 0, 0)),
            pl.BlockSpec((32, 28), lambda b: (0, 0)),
            pl.BlockSpec((32, 289), lambda b: (0, 0)),
            pl.BlockSpec((32, 289), lambda b: (0, 0)),
            pl.BlockSpec((64, 289), lambda b: (0, 0)),
            pl.BlockSpec((64, 577), lambda b: (0, 0)),
            pl.BlockSpec((1, G * 5120), lambda b: (0, 0)),
            pl.BlockSpec((1, G * 512), lambda b: (0, 0)),
        ],
        out_specs=pl.BlockSpec((G, 8, 8, 64), lambda b: (b, 0, 0, 0)),
        scratch_shapes=[
            pltpu.VMEM((289, G * 5120), _BF16),
            pltpu.VMEM((577, G * 512), _BF16),
            pltpu.VMEM((32, G * 5296), _BF16),
            pltpu.VMEM((32, G * 1648), _BF16),
            pltpu.VMEM((32, G * 592), _BF16),
            pltpu.VMEM((64, G * 592), _BF16),
            pltpu.VMEM((G * 5120, 32), _F32),
            pltpu.VMEM((G * 1536, 32), _F32),
            pltpu.VMEM((G * 512, 64), _F32),
            pltpu.VMEM((G * 34, 48, 32), _BF16),
            pltpu.VMEM((G * 18, 32, 32), _BF16),
        ],
        compiler_params=pltpu.CompilerParams(
            dimension_semantics=("parallel",),
            vmem_limit_bytes=48 * 1024 * 1024,
        ),
    )(xf, wt1, wt12, wt2, wt3, wt4, m80, m32)

    flat = feats.reshape(B, 4096)
    TM = 256 if B % 256 == 0 else B
    out = pl.pallas_call(
        _mlp_kernel,
        out_shape=jax.ShapeDtypeStruct((B, 10), _F32),
        grid=(B // TM,),
        in_specs=[
            pl.BlockSpec((TM, 4096), lambda i: (i, 0)),
            pl.BlockSpec((4096, 256), lambda i: (0, 0)),
            pl.BlockSpec((1, 256), lambda i: (0, 0)),
            pl.BlockSpec((256, 256), lambda i: (0, 0)),
            pl.BlockSpec((1, 256), lambda i: (0, 0)),
            pl.BlockSpec((256, 10), lambda i: (0, 0)),
            pl.BlockSpec((1, 10), lambda i: (0, 0)),
        ],
        out_specs=pl.BlockSpec((TM, 10), lambda i: (i, 0)),
        compiler_params=pltpu.CompilerParams(
            dimension_semantics=("parallel",),
        ),
    )(flat, lw, lb, fw, fb, ow, ob)
    return out


# speedup vs baseline: 5.4654x; 1.1483x over previous
import jax
import jax.numpy as jnp
from jax.experimental import pallas as pl
from jax.experimental.pallas import tpu as pltpu

_F32 = jnp.float32
_BF16 = jnp.bfloat16


def _make_conv_kernel(G):
    def conv_kernel(x_ref, wt1_ref, wt12_ref, wt2_ref, wt3_ref, wt4_ref,
                    m80_ref, m32_ref, o_ref,
                    rhs, rhs4, x12, x2, x3, x4, p2, p3):
        def load_taps(r, getplane, cin, stride, seg, n):
            for g in range(G):
                xv = getplane(g)
                for t in range(9):
                    dh, dw = divmod(t, 3)
                    off = g * seg + dh * stride + dw
                    r[t * cin:(t + 1) * cin, g * n:(g + 1) * n] = \
                        xv[:, off:off + n]
            r[9 * cin:9 * cin + 1, 0:G * n] = jnp.ones((1, G * n), _BF16)

        def pool_pairs(z, blk, ww):
            t = jnp.transpose(jnp.maximum(z, 0.0).astype(_BF16))
            ti = pltpu.bitcast(t, jnp.int32)
            lo = jax.lax.bitcast_convert_type(ti.astype(jnp.int16), _BF16)
            hi = jax.lax.bitcast_convert_type(
                jax.lax.shift_right_logical(ti, 16).astype(jnp.int16), _BF16)
            a = jnp.maximum(lo, hi)
            m = a.astype(_F32).reshape(blk, 2, ww, a.shape[-1])
            return jnp.maximum(m[:, 0], m[:, 1])

        xall = x_ref[...]
        load_taps(rhs, lambda g: xall[g], 3, 80, 0, 5120)
        z = jnp.dot(wt1_ref[...], rhs[0:28, :], preferred_element_type=_F32)
        y1 = jnp.maximum(z, 0.0) * m80_ref[...]
        yb = y1.astype(_BF16)
        for g in range(G):
            q = g * 5296
            x12[:, q + 81:q + 5201] = yb[:, g * 5120:(g + 1) * 5120]
            x12[:, q:q + 81] = jnp.zeros((32, 81), _BF16)
            x12[:, q + 5201:q + 5296] = jnp.zeros((32, 95), _BF16)

        xv12 = x12[...]
        load_taps(rhs, lambda g: xv12, 32, 80, 5296, 5120)
        z = jnp.dot(wt12_ref[...], rhs[0:289, :], preferred_element_type=_F32)
        pooled = pool_pairs(z, G * 32, 40)[:, 0:32, :]
        p2[...] = jnp.zeros_like(p2)
        pb = pooled.astype(_BF16)
        for g in range(G):
            p2[g * 34 + 1:g * 34 + 33, 1:33, :] = pb[g * 32:(g + 1) * 32]
        t2 = jnp.transpose(p2[...].reshape(G * 1632, 32))
        for g in range(G):
            x2[:, g * 1648:g * 1648 + 1632] = t2[:, g * 1632:(g + 1) * 1632]
            x2[:, g * 1648 + 1632:(g + 1) * 1648] = jnp.zeros((32, 16), _BF16)

        xv2 = x2[...]
        load_taps(rhs, lambda g: xv2, 32, 48, 1648, 1536)
        z = jnp.dot(wt2_ref[...], rhs[0:289, 0:G * 1536],
                    preferred_element_type=_F32)
        pooled = pool_pairs(z, G * 16, 24)[:, 0:16, :]
        p3[...] = jnp.zeros_like(p3)
        pb = pooled.astype(_BF16)
        for g in range(G):
            p3[g * 18 + 1:g * 18 + 17, 1:17, :] = pb[g * 16:(g + 1) * 16]
        t3 = jnp.transpose(p3[...].reshape(G * 576, 32))
        for g in range(G):
            x3[:, g * 592:g * 592 + 576] = t3[:, g * 576:(g + 1) * 576]
            x3[:, g * 592 + 576:(g + 1) * 592] = jnp.zeros((32, 16), _BF16)

        xv3 = x3[...]
        load_taps(rhs4, lambda g: xv3, 32, 32, 592, 512)
        z = jnp.dot(wt3_ref[...], rhs4[0:289, :], preferred_element_type=_F32)
        y3 = jnp.maximum(z, 0.0) * m32_ref[...]
        yb = y3.astype(_BF16)
        for g in range(G):
            q = g * 592
            x4[:, q + 33:q + 545] = yb[:, g * 512:(g + 1) * 512]
            x4[:, q:q + 33] = jnp.zeros((64, 33), _BF16)
            x4[:, q + 545:q + 592] = jnp.zeros((64, 47), _BF16)

        xv4 = x4[...]
        load_taps(rhs4, lambda g: xv4, 64, 32, 592, 512)
        z = jnp.dot(wt4_ref[...], rhs4[0:577, :], preferred_element_type=_F32)
        pooled = pool_pairs(z, G * 8, 16)[:, 0:8, :]
        o_ref[...] = pooled.astype(_BF16).reshape(G, 8, 8, 64)

    return conv_kernel


def _mlp_kernel(f_ref, lw_ref, lb_ref, fw_ref, fb_ref, ow_ref, ob_ref, o_ref):
    h = jnp.dot(f_ref[...].astype(_F32), lw_ref[...],
                preferred_element_type=_F32)
    h = jnp.maximum(h + lb_ref[...], 0.0)
    h = jnp.dot(h, fw_ref[...], preferred_element_type=_F32)
    h = jnp.maximum(h + fb_ref[...], 0.0)
    o_ref[...] = jnp.dot(h, ow_ref[...], preferred_element_type=_F32) + ob_ref[...]


def _wt(w, cin, cout, b):
    m = (w.reshape(3, 3, cin, cout).transpose(3, 0, 1, 2).reshape(cout, 9 * cin))
    return jnp.concatenate([m, b.reshape(cout, 1)], axis=1).astype(_BF16)


def kernel(w1, b1, w12, b12, w2, b2, w3, b3, w4, b4,
           lw, lb, fw, fb, ow, ob, x):
    B = x.shape[0]
    G = 4 if B % 4 == 0 else 1
    xp = jnp.pad(x, ((0, 0), (0, 0), (1, 1), (1, 15)))
    xf = jnp.pad(xp.reshape(B, 3, 5280),
                 ((0, 0), (0, 0), (0, 16))).astype(_BF16)

    wt1 = _wt(w1.reshape(3, 3, 8, 32)[:, :, 0:3, :].reshape(3, 9, 32), 3, 32, b1)
    wt12 = _wt(w12, 32, 32, b12)
    wt2 = _wt(w2, 32, 32, b2)
    wt3 = _wt(w3, 32, 64, b3)
    wt4 = _wt(w4, 64, 64, b4)

    m80 = jnp.tile(((jnp.arange(5120) % 80) < 64).astype(_F32), (G,)
                   ).reshape(1, G * 5120)
    m32 = jnp.tile(((jnp.arange(512) % 32) < 16).astype(_F32), (G,)
                   ).reshape(1, G * 512)

    feats = pl.pallas_call(
        _make_conv_kernel(G),
        out_shape=jax.ShapeDtypeStruct((B, 8, 8, 64), _BF16),
        grid=(B // G,),
        in_specs=[
            pl.BlockSpec((G, 3, 5296), lambda b: (b, 0, 0)),
            pl.BlockSpec((32, 28), lambda b: (0, 0)),
            pl.BlockSpec((32, 289), lambda b: (0, 0)),
            pl.BlockSpec((32, 289), lambda b: (0, 0)),
            pl.BlockSpec((64, 289), lambda b: (0, 0)),
            pl.BlockSpec((64, 577), lambda b: (0, 0)),
            pl.BlockSpec((1, G * 5120), lambda b: (0, 0)),
            pl.BlockSpec((1, G * 512), lambda b: (0, 0)),
        ],
        out_specs=pl.BlockSpec((G, 8, 8, 64), lambda b: (b, 0, 0, 0)),
        scratch_shapes=[
            pltpu.VMEM((289, G * 5120), _BF16),
            pltpu.VMEM((577, G * 512), _BF16),
            pltpu.VMEM((32, G * 5296), _BF16),
            pltpu.VMEM((32, G * 1648), _BF16),
            pltpu.VMEM((32, G * 592), _BF16),
            pltpu.VMEM((64, G * 592), _BF16),
            pltpu.VMEM((G * 34, 48, 32), _BF16),
            pltpu.VMEM((G * 18, 32, 32), _BF16),
        ],
        compiler_params=pltpu.CompilerParams(
            dimension_semantics=("parallel",),
            vmem_limit_bytes=48 * 1024 * 1024,
        ),
    )(xf, wt1, wt12, wt2, wt3, wt4, m80, m32)

    flat = feats.reshape(B, 4096)
    TM = 256 if B % 256 == 0 else B
    out = pl.pallas_call(
        _mlp_kernel,
        out_shape=jax.ShapeDtypeStruct((B, 10), _F32),
        grid=(B // TM,),
        in_specs=[
            pl.BlockSpec((TM, 4096), lambda i: (i, 0)),
            pl.BlockSpec((4096, 256), lambda i: (0, 0)),
            pl.BlockSpec((1, 256), lambda i: (0, 0)),
            pl.BlockSpec((256, 256), lambda i: (0, 0)),
            pl.BlockSpec((1, 256), lambda i: (0, 0)),
            pl.BlockSpec((256, 10), lambda i: (0, 0)),
            pl.BlockSpec((1, 10), lambda i: (0, 0)),
        ],
        out_specs=pl.BlockSpec((TM, 10), lambda i: (i, 0)),
        compiler_params=pltpu.CompilerParams(
            dimension_semantics=("parallel",),
        ),
    )(flat, lw, lb, fw, fb, ow, ob)
    return out
